# baseline (speedup 1.0000x reference)
import functools

import jax
import jax.numpy as jnp
from jax import lax
from jax.experimental import pallas as pl
from jax.experimental.pallas import tpu as pltpu
from jax.experimental.pallas import tpu_sc as plsc

_VOCAB = 1_000_000
_DIM = 32
_B = 4096
_CTX = 10
_NEG = 20
_NNEG = _CTX * _NEG
_L = 16

_info = plsc.get_sparse_core_info()
_NC = _info.num_cores
_NS = _info.num_subcores
_NW = _NC * _NS

_B_PT = _B // _NW
_O_PT = _B_PT * _CTX
_N_PT = _B_PT * _NNEG

_G = 4
_NCHK = _B_PT // _G
_NROW = _G * _NNEG
_OROW = _G * _CTX

_NGRP = _NNEG // _L
_NPAD = (_NGRP + 1) * _L
_TS = _L + 1


def _sc_score_body(owords, nwords, iv, emb_out,
                   osc_out, nsc_out,
                   iv_rows, oidx, nidx0, nidx1, rows_o0, rows_o1,
                   rows_n0, rows_n1, pscr,
                   osc_buf, nsc_buf, sem0, sem1):
    wid = lax.axis_index("s") * _NC + lax.axis_index("c")
    b0 = wid * _B_PT
    pltpu.sync_copy(iv.at[pl.ds(b0, _B_PT)], iv_rows)
    pltpu.sync_copy(owords.at[pl.ds(wid * _O_PT, _O_PT)], oidx)
    lanes17 = lax.iota(jnp.int32, _L) * _TS

    def dot16(rows, rbase, ivlo, ivhi, region):
        off = region * (_L * _TS)
        for r in range(_L):
            lo = rows[rbase + r, pl.ds(0, _L)]
            hi = rows[rbase + r, pl.ds(_L, _L)]
            p = lo * ivlo + hi * ivhi
            plsc.store_scatter(pscr, [lanes17 + (off + r)], p)
        qs = [pscr[pl.ds(off + d * _TS, _L)] for d in range(_L)]
        while len(qs) > 1:
            qs = [qs[i] + qs[i + 1] for i in range(0, len(qs), 2)]
        return qs[0]

    def issue(c, nidx_s, rows_n_s, rows_o_s, sem_s):
        pltpu.sync_copy(nwords.at[pl.ds(wid * _N_PT + c * _NROW, _NROW)],
                        nidx_s)
        pltpu.async_copy(emb_out.at[nidx_s],
                         rows_n_s.at[pl.ds(0, _NROW)], sem_s)
        pltpu.async_copy(emb_out.at[oidx.at[pl.ds(c * _OROW, _OROW)]],
                         rows_o_s.at[pl.ds(0, _OROW)], sem_s)

    def drain(c, nidx_s, rows_n_s, rows_o_s, sem_s):
        pltpu.make_async_copy(emb_out.at[nidx_s],
                              rows_n_s.at[pl.ds(0, _NROW)], sem_s).wait()
        pltpu.make_async_copy(emb_out.at[oidx.at[pl.ds(c * _OROW, _OROW)]],
                              rows_o_s.at[pl.ds(0, _OROW)], sem_s).wait()

    def compute(c, rows_n_s, rows_o_s):
        def one_b(j, carry2):
            bl = c * _G + j
            ivlo = iv_rows[bl, pl.ds(0, _L)]
            ivhi = iv_rows[bl, pl.ds(_L, _L)]
            osc_buf[bl, :] = dot16(rows_o_s, j * _CTX, ivlo, ivhi, 0)
            for g in range(_NGRP + 1):
                nsc_buf[bl, pl.ds(g * _L, _L)] = dot16(
                    rows_n_s, j * _NNEG + g * _L, ivlo, ivhi, g + 1)
            return carry2

        lax.fori_loop(0, _G, one_b, 0)

    slot0 = (nidx0, rows_n0, rows_o0, sem0)
    slot1 = (nidx1, rows_n1, rows_o1, sem1)
    issue(0, *slot0)

    def pair(k, carry):
        c0 = 2 * k
        issue(c0 + 1, *slot1)
        drain(c0, *slot0)
        compute(c0, rows_n0, rows_o0)

        @pl.when(c0 + 2 < _NCHK)
        def _():
            issue(c0 + 2, *slot0)

        drain(c0 + 1, *slot1)
        compute(c0 + 1, rows_n1, rows_o1)
        return carry

    lax.fori_loop(0, _NCHK // 2, pair, 0)
    pltpu.sync_copy(osc_buf, osc_out.at[pl.ds(b0, _B_PT)])
    pltpu.sync_copy(nsc_buf, nsc_out.at[pl.ds(b0, _B_PT)])


_sc_score = functools.partial(
    pl.kernel,
    mesh=plsc.VectorSubcoreMesh(core_axis_name="c", subcore_axis_name="s"),
    compiler_params=pltpu.CompilerParams(use_tc_tiling_on_sc=False,
                                         needs_layout_passes=False),
    out_type=[
        jax.ShapeDtypeStruct((_B, _L), jnp.float32),
        jax.ShapeDtypeStruct((_B, _NPAD), jnp.float32),
    ],
    scratch_types=[
        pltpu.VMEM((_B_PT, _DIM), jnp.float32),
        pltpu.VMEM((_O_PT,), jnp.int32),
        pltpu.VMEM((_NROW,), jnp.int32),
        pltpu.VMEM((_NROW,), jnp.int32),
        pltpu.VMEM((_OROW + _L, _DIM), jnp.float32),
        pltpu.VMEM((_OROW + _L, _DIM), jnp.float32),
        pltpu.VMEM((_NROW + _L, _DIM), jnp.float32),
        pltpu.VMEM((_NROW + _L, _DIM), jnp.float32),
        pltpu.VMEM(((_NGRP + 2) * _L * _TS,), jnp.float32),
        pltpu.VMEM((_B_PT, _L), jnp.float32),
        pltpu.VMEM((_B_PT, _NPAD), jnp.float32),
        pltpu.SemaphoreType.DMA,
        pltpu.SemaphoreType.DMA,
    ],
)(_sc_score_body)


def _loss_body(osc_ref, nsc_ref, out_ref):
    osc = osc_ref[...]
    nsc = nsc_ref[...]

    def softplus(x):
        return jnp.maximum(x, 0.0) + jnp.log(1.0 + jnp.exp(-jnp.abs(x)))

    ocol = lax.broadcasted_iota(jnp.int32, osc.shape, 1)
    ncol = lax.broadcasted_iota(jnp.int32, nsc.shape, 1)
    sp_o = jnp.where(ocol < _CTX, softplus(-osc), 0.0)
    sp_n = jnp.where(ncol < _NNEG, softplus(nsc), 0.0)
    out_ref[0, 0] = jnp.sum(sp_o) + jnp.sum(sp_n)


def _tc_loss(osc, nsc):
    out = pl.pallas_call(
        _loss_body,
        out_specs=pl.BlockSpec(memory_space=pltpu.SMEM),
        out_shape=jax.ShapeDtypeStruct((1, 1), jnp.float32),
    )(osc, nsc)
    return out[0, 0] / (_B * _CTX)


def kernel(iword, owords, nwords, emb_in, emb_out):
    iv = jnp.take(emb_in, iword, axis=0)
    osc, nsc = _sc_score(owords.reshape(-1), nwords.reshape(-1), iv, emb_out)
    return _tc_loss(osc, nsc)


# speedup vs baseline: 1.5617x; 1.5617x over previous
import functools

import jax
import jax.numpy as jnp
from jax import lax
from jax.experimental import pallas as pl
from jax.experimental.pallas import tpu as pltpu
from jax.experimental.pallas import tpu_sc as plsc

_VOCAB = 1_000_000
_DIM = 32
_B = 4096
_CTX = 10
_NEG = 20
_NNEG = _CTX * _NEG
_L = 16

_info = plsc.get_sparse_core_info()
_NC = _info.num_cores
_NS = _info.num_subcores
_NW = _NC * _NS

_B_PT = _B // _NW
_O_PT = _B_PT * _CTX
_N_PT = _B_PT * _NNEG

_G = 4
_NCHK = _B_PT // _G
_NROW = _G * _NNEG
_OROW = _G * _CTX

_NGRP = _NNEG // _L
_NPAD = (_NGRP + 1) * _L
_TS = _L + 1


def _sc_score_body(owords, nwords, iv, emb_out,
                   osc_out, nsc_out,
                   iv_rows, oidx, nidx0, nidx1, rows_o0, rows_o1,
                   rows_n0, rows_n1,
                   osc_buf, nsc_buf, sem0, sem1):
    wid = lax.axis_index("s") * _NC + lax.axis_index("c")
    b0 = wid * _B_PT
    pltpu.sync_copy(iv.at[pl.ds(b0, _B_PT)], iv_rows)
    pltpu.sync_copy(owords.at[pl.ds(wid * _O_PT, _O_PT)], oidx)
    lane = lax.iota(jnp.int32, _L)
    perms = {k: lane ^ k for k in (8, 4, 2, 1)}
    masks = {k: (lane & k) == 0 for k in (8, 4, 2, 1)}
    bitrev = [0, 8, 4, 12, 2, 10, 6, 14, 1, 9, 5, 13, 3, 11, 7, 15]

    def dot16(rows, rbase, ivlo, ivhi, region):
        del region
        vecs = []
        for i in range(_L):
            r = rbase + bitrev[i]
            lo = rows[r, pl.ds(0, _L)]
            hi = rows[r, pl.ds(_L, _L)]
            vecs.append(lo * ivlo + hi * ivhi)
        for k in (8, 4, 2, 1):
            perm, msk = perms[k], masks[k]
            nxt = []
            for i in range(0, len(vecs), 2):
                fa = vecs[i] + jnp.take_along_axis(vecs[i], perm, axis=0)
                fb = (vecs[i + 1]
                      + jnp.take_along_axis(vecs[i + 1], perm, axis=0))
                nxt.append(jnp.where(msk, fa, fb))
            vecs = nxt
        return vecs[0]

    def issue(c, nidx_s, rows_n_s, rows_o_s, sem_s):
        pltpu.sync_copy(nwords.at[pl.ds(wid * _N_PT + c * _NROW, _NROW)],
                        nidx_s)
        pltpu.async_copy(emb_out.at[nidx_s],
                         rows_n_s.at[pl.ds(0, _NROW)], sem_s)
        pltpu.async_copy(emb_out.at[oidx.at[pl.ds(c * _OROW, _OROW)]],
                         rows_o_s.at[pl.ds(0, _OROW)], sem_s)

    def drain(c, nidx_s, rows_n_s, rows_o_s, sem_s):
        pltpu.make_async_copy(emb_out.at[nidx_s],
                              rows_n_s.at[pl.ds(0, _NROW)], sem_s).wait()
        pltpu.make_async_copy(emb_out.at[oidx.at[pl.ds(c * _OROW, _OROW)]],
                              rows_o_s.at[pl.ds(0, _OROW)], sem_s).wait()

    def compute(c, rows_n_s, rows_o_s):
        def one_b(j, carry2):
            bl = c * _G + j
            ivlo = iv_rows[bl, pl.ds(0, _L)]
            ivhi = iv_rows[bl, pl.ds(_L, _L)]
            osc_buf[bl, :] = dot16(rows_o_s, j * _CTX, ivlo, ivhi, 0)
            for g in range(_NGRP + 1):
                nsc_buf[bl, pl.ds(g * _L, _L)] = dot16(
                    rows_n_s, j * _NNEG + g * _L, ivlo, ivhi, g + 1)
            return carry2

        lax.fori_loop(0, _G, one_b, 0)

    slot0 = (nidx0, rows_n0, rows_o0, sem0)
    slot1 = (nidx1, rows_n1, rows_o1, sem1)
    issue(0, *slot0)

    def pair(k, carry):
        c0 = 2 * k
        issue(c0 + 1, *slot1)
        drain(c0, *slot0)
        compute(c0, rows_n0, rows_o0)

        @pl.when(c0 + 2 < _NCHK)
        def _():
            issue(c0 + 2, *slot0)

        drain(c0 + 1, *slot1)
        compute(c0 + 1, rows_n1, rows_o1)
        return carry

    lax.fori_loop(0, _NCHK // 2, pair, 0)
    pltpu.sync_copy(osc_buf, osc_out.at[pl.ds(b0, _B_PT)])
    pltpu.sync_copy(nsc_buf, nsc_out.at[pl.ds(b0, _B_PT)])


_sc_score = functools.partial(
    pl.kernel,
    mesh=plsc.VectorSubcoreMesh(core_axis_name="c", subcore_axis_name="s"),
    compiler_params=pltpu.CompilerParams(use_tc_tiling_on_sc=False,
                                         needs_layout_passes=False),
    out_type=[
        jax.ShapeDtypeStruct((_B, _L), jnp.float32),
        jax.ShapeDtypeStruct((_B, _NPAD), jnp.float32),
    ],
    scratch_types=[
        pltpu.VMEM((_B_PT, _DIM), jnp.float32),
        pltpu.VMEM((_O_PT,), jnp.int32),
        pltpu.VMEM((_NROW,), jnp.int32),
        pltpu.VMEM((_NROW,), jnp.int32),
        pltpu.VMEM((_OROW + _L, _DIM), jnp.float32),
        pltpu.VMEM((_OROW + _L, _DIM), jnp.float32),
        pltpu.VMEM((_NROW + _L, _DIM), jnp.float32),
        pltpu.VMEM((_NROW + _L, _DIM), jnp.float32),
        pltpu.VMEM((_B_PT, _L), jnp.float32),
        pltpu.VMEM((_B_PT, _NPAD), jnp.float32),
        pltpu.SemaphoreType.DMA,
        pltpu.SemaphoreType.DMA,
    ],
)(_sc_score_body)


def _loss_body(osc_ref, nsc_ref, out_ref):
    osc = osc_ref[...]
    nsc = nsc_ref[...]

    def softplus(x):
        return jnp.maximum(x, 0.0) + jnp.log(1.0 + jnp.exp(-jnp.abs(x)))

    ocol = lax.broadcasted_iota(jnp.int32, osc.shape, 1)
    ncol = lax.broadcasted_iota(jnp.int32, nsc.shape, 1)
    sp_o = jnp.where(ocol < _CTX, softplus(-osc), 0.0)
    sp_n = jnp.where(ncol < _NNEG, softplus(nsc), 0.0)
    out_ref[0, 0] = jnp.sum(sp_o) + jnp.sum(sp_n)


def _tc_loss(osc, nsc):
    out = pl.pallas_call(
        _loss_body,
        out_specs=pl.BlockSpec(memory_space=pltpu.SMEM),
        out_shape=jax.ShapeDtypeStruct((1, 1), jnp.float32),
    )(osc, nsc)
    return out[0, 0] / (_B * _CTX)


def kernel(iword, owords, nwords, emb_in, emb_out):
    iv = jnp.take(emb_in, iword, axis=0)
    osc, nsc = _sc_score(owords.reshape(-1), nwords.reshape(-1), iv, emb_out)
    return _tc_loss(osc, nsc)
